# baseline (speedup 1.0000x reference)
import functools

import jax
import jax.numpy as jnp
from jax.experimental import pallas as pl
from jax.experimental.pallas import tpu as pltpu

DIM = 1024
HIDDEN = 4096
NUM_EXPERTS = 8
TOP_K = 2
B = 4
S = 2048

TS = 512
NS = S // TS
HC = 1024
NC = HIDDEN // HC


def _routing_kernel(x_ref, wg_ref, bg_ref, wn_ref, bn_ref, gn_ref,
                    idx_ref, wts_ref, ri_ref):
  b = pl.program_id(0)
  ri_ref[pl.ds(b, 1), :] = jnp.sum(x_ref[0], axis=0, keepdims=True) / S

  @pl.when(b == B - 1)
  def _():
    ri = ri_ref[...]
    logits = jnp.dot(ri, wg_ref[...],
                     preferred_element_type=jnp.float32) + bg_ref[...]
    nlog = jnp.dot(ri, wn_ref[...],
                   preferred_element_type=jnp.float32) + bn_ref[...]
    sp = jnp.logaddexp(nlog, 0.0)
    noisy = logits + gn_ref[...] * sp

    cols = jax.lax.broadcasted_iota(jnp.int32, (B, NUM_EXPERTS), 1)
    v0 = jnp.max(noisy, axis=1, keepdims=True)
    i0 = jnp.min(jnp.where(noisy == v0, cols, NUM_EXPERTS),
                 axis=1, keepdims=True)
    masked = jnp.where(cols == i0, -jnp.inf, noisy)
    v1 = jnp.max(masked, axis=1, keepdims=True)
    i1 = jnp.min(jnp.where(masked == v1, cols, NUM_EXPERTS),
                 axis=1, keepdims=True)
    w0 = 1.0 / (1.0 + jnp.exp(v1 - v0))
    w1 = 1.0 / (1.0 + jnp.exp(v0 - v1))

    idx_ref[:, 0:1] = i0
    idx_ref[:, 1:2] = i1
    wts_ref[:, 0:1] = w0
    wts_ref[:, 1:2] = w1


def _ffn_kernel(idx_ref, wts_ref, x_ref, w1_ref, w2_ref, b1_ref, b2_ref,
                out_ref, xbf_ref):
  b = pl.program_id(0)
  k = pl.program_id(1)
  s = pl.program_id(2)
  c = pl.program_id(3)

  e = idx_ref[b, k]
  dup = jnp.zeros((), dtype=jnp.bool_)
  for rp in range(B * TOP_K):
    bp, kp = rp // TOP_K, rp % TOP_K
    earlier = rp < b * TOP_K + k
    dup = jnp.logical_or(dup,
                         jnp.logical_and(earlier, idx_ref[bp, kp] == e))
  keep = jnp.logical_not(dup)

  row = pl.ds(s * TS, TS)
  first = jnp.logical_and(k == 0, c == 0)

  @pl.when(jnp.logical_and(first, jnp.logical_not(keep)))
  def _():
    out_ref[0, row, :] = jnp.zeros((TS, DIM), dtype=jnp.float32)

  @pl.when(keep)
  def _():
    @pl.when(c == 0)
    def _():
      xbf_ref[...] = x_ref[0].astype(jnp.bfloat16)

    w = wts_ref[b, k]
    h = jnp.dot(xbf_ref[...], w1_ref[0].astype(jnp.bfloat16),
                preferred_element_type=jnp.float32) + b1_ref[0]
    h = 0.5 * h * (1.0 + jax.lax.erf(h * 0.7071067811865476))
    yc = jnp.dot(h.astype(jnp.bfloat16), w2_ref[0].astype(jnp.bfloat16),
                 preferred_element_type=jnp.float32)
    contrib = jnp.where(c == 0, yc + b2_ref[0], yc) * w

    @pl.when(first)
    def _():
      out_ref[0, row, :] = contrib

    @pl.when(jnp.logical_not(first))
    def _():
      out_ref[0, row, :] = out_ref[0, row, :] + contrib


@jax.jit
def kernel(router_input, x, Wg, bg, Wn, bn, W1, b1, W2, b2):
  gnoise = jax.random.normal(jax.random.key(42), (B, NUM_EXPERTS),
                             dtype=jnp.float32)

  idx, wts = pl.pallas_call(
      _routing_kernel,
      grid=(B,),
      in_specs=[
          pl.BlockSpec((1, S, DIM), lambda b: (b, 0, 0)),
          pl.BlockSpec((DIM, NUM_EXPERTS), lambda b: (0, 0)),
          pl.BlockSpec((1, NUM_EXPERTS), lambda b: (0, 0)),
          pl.BlockSpec((DIM, NUM_EXPERTS), lambda b: (0, 0)),
          pl.BlockSpec((1, NUM_EXPERTS), lambda b: (0, 0)),
          pl.BlockSpec((B, NUM_EXPERTS), lambda b: (0, 0)),
      ],
      out_specs=[
          pl.BlockSpec((B, TOP_K), lambda b: (0, 0)),
          pl.BlockSpec((B, TOP_K), lambda b: (0, 0)),
      ],
      out_shape=[
          jax.ShapeDtypeStruct((B, TOP_K), jnp.int32),
          jax.ShapeDtypeStruct((B, TOP_K), jnp.float32),
      ],
      scratch_shapes=[pltpu.MemorySpace.VMEM((B, DIM), jnp.float32)],
      compiler_params=pltpu.CompilerParams(
          dimension_semantics=("arbitrary",)),
  )(router_input, Wg, bg.reshape(1, -1), Wn, bn.reshape(1, -1), gnoise)

  b1r = b1.reshape(NUM_EXPERTS, 1, HIDDEN)
  b2r = b2.reshape(NUM_EXPERTS, 1, DIM)

  grid_spec = pltpu.PrefetchScalarGridSpec(
      num_scalar_prefetch=1,
      grid=(B, TOP_K, NS, NC),
      in_specs=[
          pl.BlockSpec((B, TOP_K), memory_space=pltpu.SMEM),
          pl.BlockSpec((1, TS, DIM), lambda b, k, s, c, idx: (b, s, 0)),
          pl.BlockSpec((1, DIM, HC),
                       lambda b, k, s, c, idx: (idx[b, k], 0, c)),
          pl.BlockSpec((1, HC, DIM),
                       lambda b, k, s, c, idx: (idx[b, k], c, 0)),
          pl.BlockSpec((1, 1, HC),
                       lambda b, k, s, c, idx: (idx[b, k], 0, c)),
          pl.BlockSpec((1, 1, DIM),
                       lambda b, k, s, c, idx: (idx[b, k], 0, 0)),
      ],
      out_specs=pl.BlockSpec((1, S, DIM), lambda b, k, s, c, idx: (b, 0, 0)),
      scratch_shapes=[pltpu.MemorySpace.VMEM((TS, DIM), jnp.bfloat16)],
  )

  out = pl.pallas_call(
      _ffn_kernel,
      grid_spec=grid_spec,
      out_shape=jax.ShapeDtypeStruct((B, S, DIM), jnp.float32),
      compiler_params=pltpu.CompilerParams(
          dimension_semantics=("parallel",) + ("arbitrary",) * 3),
  )(idx, wts, x, W1, W2, b1r, b2r)

  return out


# speedup vs baseline: 1.0931x; 1.0931x over previous
import functools

import jax
import jax.numpy as jnp
from jax.experimental import pallas as pl
from jax.experimental.pallas import tpu as pltpu

DIM = 1024
HIDDEN = 4096
NUM_EXPERTS = 8
TOP_K = 2
B = 4
S = 2048

TS = 512
NS = S // TS
HC = 2048
NC = HIDDEN // HC


def _routing_kernel(x_ref, wg_ref, bg_ref, wn_ref, bn_ref, gn_ref,
                    idx_ref, wts_ref, ri_ref):
  b = pl.program_id(0)
  ri_ref[pl.ds(b, 1), :] = jnp.sum(x_ref[0], axis=0, keepdims=True) / S

  @pl.when(b == B - 1)
  def _():
    ri = ri_ref[...]
    logits = jnp.dot(ri, wg_ref[...],
                     preferred_element_type=jnp.float32) + bg_ref[...]
    nlog = jnp.dot(ri, wn_ref[...],
                   preferred_element_type=jnp.float32) + bn_ref[...]
    sp = jnp.logaddexp(nlog, 0.0)
    noisy = logits + gn_ref[...] * sp

    cols = jax.lax.broadcasted_iota(jnp.int32, (B, NUM_EXPERTS), 1)
    v0 = jnp.max(noisy, axis=1, keepdims=True)
    i0 = jnp.min(jnp.where(noisy == v0, cols, NUM_EXPERTS),
                 axis=1, keepdims=True)
    masked = jnp.where(cols == i0, -jnp.inf, noisy)
    v1 = jnp.max(masked, axis=1, keepdims=True)
    i1 = jnp.min(jnp.where(masked == v1, cols, NUM_EXPERTS),
                 axis=1, keepdims=True)
    w0 = 1.0 / (1.0 + jnp.exp(v1 - v0))
    w1 = 1.0 / (1.0 + jnp.exp(v0 - v1))

    idx_ref[:, 0:1] = i0
    idx_ref[:, 1:2] = i1
    wts_ref[:, 0:1] = w0
    wts_ref[:, 1:2] = w1


def _ffn_kernel(idx_ref, wts_ref, x_ref, w1_ref, w2_ref, b1_ref, b2_ref,
                out_ref):
  b = pl.program_id(0)
  k = pl.program_id(1)
  s = pl.program_id(2)
  c = pl.program_id(3)

  e = idx_ref[b, k]
  dup = jnp.zeros((), dtype=jnp.bool_)
  for rp in range(B * TOP_K):
    bp, kp = rp // TOP_K, rp % TOP_K
    earlier = rp < b * TOP_K + k
    dup = jnp.logical_or(dup,
                         jnp.logical_and(earlier, idx_ref[bp, kp] == e))
  keep = jnp.logical_not(dup)

  row = pl.ds(s * TS, TS)
  first = jnp.logical_and(k == 0, c == 0)

  @pl.when(jnp.logical_and(first, jnp.logical_not(keep)))
  def _():
    out_ref[0, row, :] = jnp.zeros((TS, DIM), dtype=jnp.float32)

  @pl.when(keep)
  def _():
    w = wts_ref[b, k]
    h = jnp.dot(x_ref[0], w1_ref[0],
                preferred_element_type=jnp.float32) + b1_ref[0]
    h = 0.5 * h * (1.0 + jax.lax.erf(h * 0.7071067811865476))
    yc = jnp.dot(h, w2_ref[0],
                 preferred_element_type=jnp.float32)
    contrib = jnp.where(c == 0, yc + b2_ref[0], yc) * w

    @pl.when(first)
    def _():
      out_ref[0, row, :] = contrib

    @pl.when(jnp.logical_not(first))
    def _():
      out_ref[0, row, :] = out_ref[0, row, :] + contrib


@jax.jit
def kernel(router_input, x, Wg, bg, Wn, bn, W1, b1, W2, b2):
  gnoise = jax.random.normal(jax.random.key(42), (B, NUM_EXPERTS),
                             dtype=jnp.float32)

  idx, wts = pl.pallas_call(
      _routing_kernel,
      grid=(B,),
      in_specs=[
          pl.BlockSpec((1, S, DIM), lambda b: (b, 0, 0)),
          pl.BlockSpec((DIM, NUM_EXPERTS), lambda b: (0, 0)),
          pl.BlockSpec((1, NUM_EXPERTS), lambda b: (0, 0)),
          pl.BlockSpec((DIM, NUM_EXPERTS), lambda b: (0, 0)),
          pl.BlockSpec((1, NUM_EXPERTS), lambda b: (0, 0)),
          pl.BlockSpec((B, NUM_EXPERTS), lambda b: (0, 0)),
      ],
      out_specs=[
          pl.BlockSpec((B, TOP_K), lambda b: (0, 0)),
          pl.BlockSpec((B, TOP_K), lambda b: (0, 0)),
      ],
      out_shape=[
          jax.ShapeDtypeStruct((B, TOP_K), jnp.int32),
          jax.ShapeDtypeStruct((B, TOP_K), jnp.float32),
      ],
      scratch_shapes=[pltpu.MemorySpace.VMEM((B, DIM), jnp.float32)],
      compiler_params=pltpu.CompilerParams(
          dimension_semantics=("arbitrary",)),
  )(router_input, Wg, bg.reshape(1, -1), Wn, bn.reshape(1, -1), gnoise)

  b1r = b1.reshape(NUM_EXPERTS, 1, HIDDEN)
  b2r = b2.reshape(NUM_EXPERTS, 1, DIM)

  grid_spec = pltpu.PrefetchScalarGridSpec(
      num_scalar_prefetch=1,
      grid=(B, TOP_K, NS, NC),
      in_specs=[
          pl.BlockSpec((B, TOP_K), memory_space=pltpu.SMEM),
          pl.BlockSpec((1, TS, DIM), lambda b, k, s, c, idx: (b, s, 0)),
          pl.BlockSpec((1, DIM, HC),
                       lambda b, k, s, c, idx: (idx[b, k], 0, c)),
          pl.BlockSpec((1, HC, DIM),
                       lambda b, k, s, c, idx: (idx[b, k], c, 0)),
          pl.BlockSpec((1, 1, HC),
                       lambda b, k, s, c, idx: (idx[b, k], 0, c)),
          pl.BlockSpec((1, 1, DIM),
                       lambda b, k, s, c, idx: (idx[b, k], 0, 0)),
      ],
      out_specs=pl.BlockSpec((1, S, DIM), lambda b, k, s, c, idx: (b, 0, 0)),
  )

  out = pl.pallas_call(
      _ffn_kernel,
      grid_spec=grid_spec,
      out_shape=jax.ShapeDtypeStruct((B, S, DIM), jnp.float32),
      compiler_params=pltpu.CompilerParams(
          dimension_semantics=("arbitrary",) * 4,
          vmem_limit_bytes=100 * 1024 * 1024),
  )(idx, wts, x, W1, W2, b1r, b2r)

  return out


# speedup vs baseline: 1.3429x; 1.2286x over previous
import functools

import jax
import jax.numpy as jnp
from jax.experimental import pallas as pl
from jax.experimental.pallas import tpu as pltpu

DIM = 1024
HIDDEN = 4096
NUM_EXPERTS = 8
TOP_K = 2
B = 4
S = 2048

TS = 512
NS = S // TS
HC = 2048
NC = HIDDEN // HC


def _routing_kernel(x_ref, wg_ref, bg_ref, wn_ref, bn_ref, gn_ref,
                    idx_ref, wts_ref, ri_ref):
  b = pl.program_id(0)
  ri_ref[pl.ds(b, 1), :] = jnp.sum(x_ref[0], axis=0, keepdims=True) / S

  @pl.when(b == B - 1)
  def _():
    ri = ri_ref[...]
    logits = jnp.dot(ri, wg_ref[...],
                     preferred_element_type=jnp.float32) + bg_ref[...]
    nlog = jnp.dot(ri, wn_ref[...],
                   preferred_element_type=jnp.float32) + bn_ref[...]
    sp = jnp.logaddexp(nlog, 0.0)
    noisy = logits + gn_ref[...] * sp

    cols = jax.lax.broadcasted_iota(jnp.int32, (B, NUM_EXPERTS), 1)
    v0 = jnp.max(noisy, axis=1, keepdims=True)
    i0 = jnp.min(jnp.where(noisy == v0, cols, NUM_EXPERTS),
                 axis=1, keepdims=True)
    masked = jnp.where(cols == i0, -jnp.inf, noisy)
    v1 = jnp.max(masked, axis=1, keepdims=True)
    i1 = jnp.min(jnp.where(masked == v1, cols, NUM_EXPERTS),
                 axis=1, keepdims=True)
    w0 = 1.0 / (1.0 + jnp.exp(v1 - v0))
    w1 = 1.0 / (1.0 + jnp.exp(v0 - v1))

    idx_ref[:, 0:1] = i0
    idx_ref[:, 1:2] = i1
    wts_ref[:, 0:1] = w0
    wts_ref[:, 1:2] = w1


def _ffn_kernel(idx_ref, wts_ref, x_ref, w1_ref, w2_ref, b1_ref, b2_ref,
                out_ref):
  b = pl.program_id(0)
  k = pl.program_id(1)
  c = pl.program_id(2)
  s = pl.program_id(3)

  e = idx_ref[b, k]
  dup = jnp.zeros((), dtype=jnp.bool_)
  for rp in range(B * TOP_K):
    bp, kp = rp // TOP_K, rp % TOP_K
    earlier = rp < b * TOP_K + k
    dup = jnp.logical_or(dup,
                         jnp.logical_and(earlier, idx_ref[bp, kp] == e))
  keep = jnp.logical_not(dup)

  row = pl.ds(s * TS, TS)
  first = jnp.logical_and(k == 0, c == 0)

  @pl.when(jnp.logical_and(first, jnp.logical_not(keep)))
  def _():
    out_ref[0, row, :] = jnp.zeros((TS, DIM), dtype=jnp.float32)

  @pl.when(keep)
  def _():
    w = wts_ref[b, k]
    h = jnp.dot(x_ref[0], w1_ref[0],
                preferred_element_type=jnp.float32) + b1_ref[0]
    h = 0.5 * h * (1.0 + jax.lax.erf(h * 0.7071067811865476))
    yc = jnp.dot(h, w2_ref[0],
                 preferred_element_type=jnp.float32)
    contrib = jnp.where(c == 0, yc + b2_ref[0], yc) * w

    @pl.when(first)
    def _():
      out_ref[0, row, :] = contrib

    @pl.when(jnp.logical_not(first))
    def _():
      out_ref[0, row, :] = out_ref[0, row, :] + contrib


@jax.jit
def kernel(router_input, x, Wg, bg, Wn, bn, W1, b1, W2, b2):
  gnoise = jax.random.normal(jax.random.key(42), (B, NUM_EXPERTS),
                             dtype=jnp.float32)

  idx, wts = pl.pallas_call(
      _routing_kernel,
      grid=(B,),
      in_specs=[
          pl.BlockSpec((1, S, DIM), lambda b: (b, 0, 0)),
          pl.BlockSpec((DIM, NUM_EXPERTS), lambda b: (0, 0)),
          pl.BlockSpec((1, NUM_EXPERTS), lambda b: (0, 0)),
          pl.BlockSpec((DIM, NUM_EXPERTS), lambda b: (0, 0)),
          pl.BlockSpec((1, NUM_EXPERTS), lambda b: (0, 0)),
          pl.BlockSpec((B, NUM_EXPERTS), lambda b: (0, 0)),
      ],
      out_specs=[
          pl.BlockSpec((B, TOP_K), lambda b: (0, 0)),
          pl.BlockSpec((B, TOP_K), lambda b: (0, 0)),
      ],
      out_shape=[
          jax.ShapeDtypeStruct((B, TOP_K), jnp.int32),
          jax.ShapeDtypeStruct((B, TOP_K), jnp.float32),
      ],
      scratch_shapes=[pltpu.MemorySpace.VMEM((B, DIM), jnp.float32)],
      compiler_params=pltpu.CompilerParams(
          dimension_semantics=("arbitrary",)),
  )(router_input, Wg, bg.reshape(1, -1), Wn, bn.reshape(1, -1), gnoise)

  b1r = b1.reshape(NUM_EXPERTS, 1, HIDDEN)
  b2r = b2.reshape(NUM_EXPERTS, 1, DIM)

  grid_spec = pltpu.PrefetchScalarGridSpec(
      num_scalar_prefetch=1,
      grid=(B, TOP_K, NC, NS),
      in_specs=[
          pl.BlockSpec((B, TOP_K), memory_space=pltpu.SMEM),
          pl.BlockSpec((1, TS, DIM), lambda b, k, c, s, idx: (b, s, 0)),
          pl.BlockSpec((1, DIM, HC),
                       lambda b, k, c, s, idx: (idx[b, k], 0, c)),
          pl.BlockSpec((1, HC, DIM),
                       lambda b, k, c, s, idx: (idx[b, k], c, 0)),
          pl.BlockSpec((1, 1, HC),
                       lambda b, k, c, s, idx: (idx[b, k], 0, c)),
          pl.BlockSpec((1, 1, DIM),
                       lambda b, k, c, s, idx: (idx[b, k], 0, 0)),
      ],
      out_specs=pl.BlockSpec((1, S, DIM), lambda b, k, c, s, idx: (b, 0, 0)),
  )

  out = pl.pallas_call(
      _ffn_kernel,
      grid_spec=grid_spec,
      out_shape=jax.ShapeDtypeStruct((B, S, DIM), jnp.float32),
      compiler_params=pltpu.CompilerParams(
          dimension_semantics=("arbitrary",) * 4,
          vmem_limit_bytes=100 * 1024 * 1024),
  )(idx, wts, x, W1, W2, b1r, b2r)

  return out
